# baseline (speedup 1.0000x reference)
import jax
import jax.numpy as jnp
from jax import lax
from jax.experimental import pallas as pl
from jax.experimental.pallas import tpu as pltpu
from jax.experimental.pallas import tpu_sc as plsc

E = 320000
V = 10000
D = 128
DP = D // 2
NC = 2
NS = 16
L = 16
NW = NC * NS
EW = E // NW
CHE = 125
CHB = 128
NCHUNK = EW // CHE
NBUF = 4
NGB = CHB // L
PR = 125
VS = V // NS
NPACK = VS // PR


def _body(xu, xr, eli, out, pu, pr_, idxu_v, idxr_v, urows, rrows, pbuf,
          out_all, pin, pout, sem, psem):
    c = lax.axis_index("c")
    s = lax.axis_index("s")
    wid = s * NC + c
    base_w = wid * EW

    def pack_table(src, dst):
        def pack_step(pi, carry):
            r0 = s * VS + pi * PR
            pltpu.sync_copy(src.at[pl.ds(r0, PR)], pin)
            @pl.when(pi >= 1)
            def _():
                pltpu.make_async_copy(pout, dst.at[c, pl.ds(0, PR)],
                                      psem).wait()
            def prow(i5, rcarry):
                for u in range(5):
                    i = i5 * 5 + u
                    for k in range(DP // L):
                        a = pin[i, pl.ds(k * L, L)]
                        b = pin[i, pl.ds(DP + k * L, L)]
                        w = plsc.pack(a, b, format=plsc.PackFormat.INTERLEAVED)
                        pout[i, pl.ds(k * L, L)] = plsc.bitcast(w, jnp.int32)
                return rcarry
            lax.fori_loop(0, PR // 5, prow, 0, unroll=False)
            pltpu.async_copy(pout, dst.at[c, pl.ds(r0, PR)], psem)
            return carry
        lax.fori_loop(0, NPACK, pack_step, 0, unroll=False)
        pltpu.make_async_copy(pout, dst.at[c, pl.ds(0, PR)], psem).wait()

    pack_table(xu, pu)
    pack_table(xr, pr_)
    plsc.subcore_barrier()

    pltpu.sync_copy(eli.at[0, wid], idxu_v)
    pltpu.sync_copy(eli.at[1, wid], idxr_v)

    def issue(ci, b):
        pltpu.async_copy(
            pu.at[c].at[idxu_v.at[ci]], urows.at[b], sem.at[b])
        pltpu.async_copy(
            pr_.at[c].at[idxr_v.at[ci]], rrows.at[b], sem.at[b])

    for b in range(NBUF):
        issue(b, b)

    def chunk_body(ci, carry):
        b = lax.rem(ci, NBUF)
        pltpu.make_async_copy(pu.at[0, pl.ds(0, CHB)], urows.at[b],
                              sem.at[b]).wait()
        pltpu.make_async_copy(pu.at[0, pl.ds(0, CHB)], rrows.at[b],
                              sem.at[b]).wait()

        def group_body(g, gcarry):
            for i in range(L):
                e = g * L + i
                p = None
                for k in range(DP // L):
                    uv = plsc.bitcast(urows[b, e, pl.ds(k * L, L)], jnp.bfloat16)
                    rv = plsc.bitcast(rrows[b, e, pl.ds(k * L, L)], jnp.bfloat16)
                    t = uv * rv
                    p = t if p is None else p + t
                pa, pb = plsc.unpack(p, format=plsc.PackFormat.INTERLEAVED)
                pbuf[pl.ds(i * L, L)] = pa + pb
            ebase = lax.iota(jnp.int32, L) * L
            acc = plsc.load_gather(pbuf, [ebase])
            for j in range(1, L):
                acc = acc + plsc.load_gather(pbuf, [ebase + j])
            out_all[pl.ds(ci * CHE + g * L, L)] = acc
            return gcarry

        lax.fori_loop(0, NGB, group_body, 0, unroll=False)

        @pl.when(ci < NCHUNK - NBUF)
        def _():
            issue(ci + NBUF, b)

        return carry

    lax.fori_loop(0, NCHUNK, chunk_body, 0, unroll=False)
    pltpu.sync_copy(out_all.at[pl.ds(0, EW)], out.at[pl.ds(base_w, EW)])


@jax.jit
def _run(xu, xr, eli):
    eli = jnp.pad(eli.reshape(2, NW, NCHUNK, CHE), ((0, 0),) * 3 + ((0, CHB - CHE),))
    mesh = plsc.VectorSubcoreMesh(
        core_axis_name="c", subcore_axis_name="s", num_cores=NC,
        num_subcores=NS)
    return pl.kernel(
        _body,
        out_type=[
            jax.ShapeDtypeStruct((E,), jnp.float32),
            jax.ShapeDtypeStruct((NC, V, DP), jnp.int32),
            jax.ShapeDtypeStruct((NC, V, DP), jnp.int32),
        ],
        mesh=mesh,
        compiler_params=pltpu.CompilerParams(
            needs_layout_passes=False, use_tc_tiling_on_sc=False),
        scratch_types=[
            pltpu.VMEM((NCHUNK, CHB), jnp.int32),
            pltpu.VMEM((NCHUNK, CHB), jnp.int32),
            pltpu.VMEM((NBUF, CHB, DP), jnp.int32),
            pltpu.VMEM((NBUF, CHB, DP), jnp.int32),
            pltpu.VMEM((L * L,), jnp.float32),
            pltpu.VMEM((EW + L,), jnp.float32),
            pltpu.VMEM((PR, D), jnp.float32),
            pltpu.VMEM((PR, DP), jnp.int32),
            pltpu.SemaphoreType.DMA((NBUF,)),
            pltpu.SemaphoreType.DMA,
        ],
    )(xu, xr, eli)


def kernel(x_user, x_restaurant, edge_label_index):
    out, _, _ = _run(x_user, x_restaurant,
                     edge_label_index.astype(jnp.int32))
    return out


# speedup vs baseline: 1.0462x; 1.0462x over previous
import jax
import jax.numpy as jnp
from jax import lax
from jax.experimental import pallas as pl
from jax.experimental.pallas import tpu as pltpu
from jax.experimental.pallas import tpu_sc as plsc

E = 320000
V = 10000
D = 128
DP = D // 2
NC = 2
NS = 16
L = 16
NW = NC * NS
EW = E // NW
CHB = 80
NCHUNK = EW // CHB
NBUF = 6
NGB = CHB // L
PR = 125
VS = V // NS
NPACK = VS // PR


def _body(xu, xr, eli, out, pu, pr_, idxu_v, idxr_v, urows, rrows, pbuf,
          out_all, pin, pout, sem, psem):
    c = lax.axis_index("c")
    s = lax.axis_index("s")
    wid = s * NC + c
    base_w = wid * EW

    def pack_table(src, dst):
        def pack_step(pi, carry):
            r0 = s * VS + pi * PR
            pltpu.sync_copy(src.at[pl.ds(r0, PR)], pin)
            @pl.when(pi >= 1)
            def _():
                pltpu.make_async_copy(pout, dst.at[c, pl.ds(0, PR)],
                                      psem).wait()
            def prow(i5, rcarry):
                for u in range(5):
                    i = i5 * 5 + u
                    for k in range(DP // L):
                        a = pin[i, pl.ds(k * L, L)]
                        b = pin[i, pl.ds(DP + k * L, L)]
                        w = plsc.pack(a, b, format=plsc.PackFormat.INTERLEAVED)
                        pout[i, pl.ds(k * L, L)] = plsc.bitcast(w, jnp.int32)
                return rcarry
            lax.fori_loop(0, PR // 5, prow, 0, unroll=False)
            pltpu.async_copy(pout, dst.at[c, pl.ds(r0, PR)], psem)
            return carry
        lax.fori_loop(0, NPACK, pack_step, 0, unroll=False)
        pltpu.make_async_copy(pout, dst.at[c, pl.ds(0, PR)], psem).wait()

    pack_table(xu, pu)
    pack_table(xr, pr_)
    plsc.subcore_barrier()

    pltpu.sync_copy(eli.at[0, pl.ds(base_w, EW)], idxu_v)
    pltpu.sync_copy(eli.at[1, pl.ds(base_w, EW)], idxr_v)

    def issue(ci, b):
        pltpu.async_copy(
            pu.at[c].at[idxu_v.at[pl.ds(ci * CHB, CHB)]], urows.at[b],
            sem.at[b])
        pltpu.async_copy(
            pr_.at[c].at[idxr_v.at[pl.ds(ci * CHB, CHB)]], rrows.at[b],
            sem.at[b])

    for b in range(NBUF):
        issue(b, b)

    def chunk_body(ci, carry):
        b = lax.rem(ci, NBUF)
        pltpu.make_async_copy(pu.at[0, pl.ds(0, CHB)], urows.at[b],
                              sem.at[b]).wait()
        pltpu.make_async_copy(pu.at[0, pl.ds(0, CHB)], rrows.at[b],
                              sem.at[b]).wait()

        def group_pair(h, gcarry):
            for half in range(2):
                g = h * 2 + half
                pb0 = half * L * L
                for i in range(L):
                    e = g * L + i
                    p = None
                    for k in range(DP // L):
                        uv = plsc.bitcast(urows[b, e, pl.ds(k * L, L)], jnp.bfloat16)
                        rv = plsc.bitcast(rrows[b, e, pl.ds(k * L, L)], jnp.bfloat16)
                        t = uv * rv
                        p = t if p is None else p + t
                    pa, pbv = plsc.unpack(p, format=plsc.PackFormat.INTERLEAVED)
                    pbuf[pl.ds(pb0 + i * L, L)] = pa + pbv
            ebase = lax.iota(jnp.int32, L) * L
            for half in range(2):
                g = h * 2 + half
                pb0 = half * L * L
                acc = plsc.load_gather(pbuf, [pb0 + ebase])
                for j in range(1, L):
                    acc = acc + plsc.load_gather(pbuf, [pb0 + ebase + j])
                out_all[pl.ds(ci * CHB + g * L, L)] = acc
            return gcarry

        lax.fori_loop(0, NGB // 2, group_pair, 0, unroll=False)

        def tail_group(g):
            for i in range(L):
                e = g * L + i
                p = None
                for k in range(DP // L):
                    uv = plsc.bitcast(urows[b, e, pl.ds(k * L, L)], jnp.bfloat16)
                    rv = plsc.bitcast(rrows[b, e, pl.ds(k * L, L)], jnp.bfloat16)
                    t = uv * rv
                    p = t if p is None else p + t
                pa, pbv = plsc.unpack(p, format=plsc.PackFormat.INTERLEAVED)
                pbuf[pl.ds(i * L, L)] = pa + pbv
            ebase = lax.iota(jnp.int32, L) * L
            acc = plsc.load_gather(pbuf, [ebase])
            for j in range(1, L):
                acc = acc + plsc.load_gather(pbuf, [ebase + j])
            out_all[pl.ds(ci * CHB + g * L, L)] = acc

        tail_group(NGB - 1)

        @pl.when(ci < NCHUNK - NBUF)
        def _():
            issue(ci + NBUF, b)

        return carry

    lax.fori_loop(0, NCHUNK, chunk_body, 0, unroll=False)
    pltpu.sync_copy(out_all, out.at[pl.ds(base_w, EW)])


@jax.jit
def _run(xu, xr, eli):
    mesh = plsc.VectorSubcoreMesh(
        core_axis_name="c", subcore_axis_name="s", num_cores=NC,
        num_subcores=NS)
    return pl.kernel(
        _body,
        out_type=[
            jax.ShapeDtypeStruct((E,), jnp.float32),
            jax.ShapeDtypeStruct((NC, V, DP), jnp.int32),
            jax.ShapeDtypeStruct((NC, V, DP), jnp.int32),
        ],
        mesh=mesh,
        compiler_params=pltpu.CompilerParams(
            needs_layout_passes=False, use_tc_tiling_on_sc=False),
        scratch_types=[
            pltpu.VMEM((EW,), jnp.int32),
            pltpu.VMEM((EW,), jnp.int32),
            pltpu.VMEM((NBUF, CHB, DP), jnp.int32),
            pltpu.VMEM((NBUF, CHB, DP), jnp.int32),
            pltpu.VMEM((2 * L * L,), jnp.float32),
            pltpu.VMEM((EW,), jnp.float32),
            pltpu.VMEM((PR, D), jnp.float32),
            pltpu.VMEM((PR, DP), jnp.int32),
            pltpu.SemaphoreType.DMA((NBUF,)),
            pltpu.SemaphoreType.DMA,
        ],
    )(xu, xr, eli)


def kernel(x_user, x_restaurant, edge_label_index):
    out, _, _ = _run(x_user, x_restaurant,
                     edge_label_index.astype(jnp.int32))
    return out


# speedup vs baseline: 1.1091x; 1.0601x over previous
import jax
import jax.numpy as jnp
from jax import lax
from jax.experimental import pallas as pl
from jax.experimental.pallas import tpu as pltpu
from jax.experimental.pallas import tpu_sc as plsc

E = 320000
V = 10000
D = 128
DP = D // 2
NC = 2
NS = 16
L = 16
NW = NC * NS
EW = E // NW
CHB = 80
NCHUNK = EW // CHB
NBUF = 5
NGB = CHB // L
PR = 125
VS = V // NS
NPACK = VS // PR


def _body(xu, xr, eli, out, pu, pr_, idxu_v, idxr_v, urows, rrows, pbuf,
          out_all, pin, pout, sem, insem, outsem):
    c = lax.axis_index("c")
    s = lax.axis_index("s")
    wid = s * NC + c
    base_w = wid * EW

    icp0 = pltpu.async_copy(eli.at[0, pl.ds(base_w, EW)], idxu_v, sem.at[0])
    icp1 = pltpu.async_copy(eli.at[1, pl.ds(base_w, EW)], idxr_v, sem.at[1])

    def pack_table(src, dst):
        pltpu.async_copy(src.at[pl.ds(s * VS, PR)], pin.at[0], insem.at[0])

        def pack_step(pi, carry):
            pb = lax.rem(pi, 2)
            r0 = s * VS + pi * PR
            pltpu.make_async_copy(src.at[pl.ds(0, PR)], pin.at[pb],
                                  insem.at[pb]).wait()

            @pl.when(pi + 1 < NPACK)
            def _():
                pltpu.async_copy(src.at[pl.ds(r0 + PR, PR)],
                                 pin.at[1 - pb], insem.at[1 - pb])

            @pl.when(pi >= 1)
            def _():
                pltpu.make_async_copy(pout, dst.at[c, pl.ds(0, PR)],
                                      outsem).wait()

            def prow(i5, rcarry):
                for u in range(5):
                    i = i5 * 5 + u
                    for k in range(DP // L):
                        a = pin[pb, i, pl.ds(k * L, L)]
                        b = pin[pb, i, pl.ds(DP + k * L, L)]
                        w = plsc.pack(a, b, format=plsc.PackFormat.INTERLEAVED)
                        pout[i, pl.ds(k * L, L)] = plsc.bitcast(w, jnp.int32)
                return rcarry
            lax.fori_loop(0, PR // 5, prow, 0, unroll=False)
            pltpu.async_copy(pout, dst.at[c, pl.ds(r0, PR)], outsem)
            return carry
        lax.fori_loop(0, NPACK, pack_step, 0, unroll=False)
        pltpu.make_async_copy(pout, dst.at[c, pl.ds(0, PR)],
                              outsem).wait()

    pack_table(xu, pu)
    pack_table(xr, pr_)
    plsc.subcore_barrier()

    icp0.wait()
    icp1.wait()

    def issue(ci, b):
        pltpu.async_copy(
            pu.at[c].at[idxu_v.at[pl.ds(ci * CHB, CHB)]], urows.at[b],
            sem.at[b])
        pltpu.async_copy(
            pr_.at[c].at[idxr_v.at[pl.ds(ci * CHB, CHB)]], rrows.at[b],
            sem.at[b])

    for b in range(NBUF):
        issue(b, b)

    def chunk_body(ci, carry):
        b = lax.rem(ci, NBUF)
        pltpu.make_async_copy(pu.at[0, pl.ds(0, CHB)], urows.at[b],
                              sem.at[b]).wait()
        pltpu.make_async_copy(pu.at[0, pl.ds(0, CHB)], rrows.at[b],
                              sem.at[b]).wait()

        def group_body(g, gcarry):
            for i in range(L):
                e = g * L + i
                p = None
                for k in range(DP // L):
                    uv = plsc.bitcast(urows[b, e, pl.ds(k * L, L)], jnp.bfloat16)
                    rv = plsc.bitcast(rrows[b, e, pl.ds(k * L, L)], jnp.bfloat16)
                    t = uv * rv
                    p = t if p is None else p + t
                pa, pb = plsc.unpack(p, format=plsc.PackFormat.INTERLEAVED)
                pbuf[pl.ds(i * L, L)] = pa + pb
            ebase = lax.iota(jnp.int32, L) * L
            acc = plsc.load_gather(pbuf, [ebase])
            for j in range(1, L):
                acc = acc + plsc.load_gather(pbuf, [ebase + j])
            out_all[pl.ds(ci * CHB + g * L, L)] = acc
            return gcarry

        lax.fori_loop(0, NGB, group_body, 0, unroll=False)

        @pl.when(ci < NCHUNK - NBUF)
        def _():
            issue(ci + NBUF, b)

        return carry

    lax.fori_loop(0, NCHUNK, chunk_body, 0, unroll=False)
    pltpu.sync_copy(out_all, out.at[pl.ds(base_w, EW)])


@jax.jit
def _run(xu, xr, eli):
    mesh = plsc.VectorSubcoreMesh(
        core_axis_name="c", subcore_axis_name="s", num_cores=NC,
        num_subcores=NS)
    return pl.kernel(
        _body,
        out_type=[
            jax.ShapeDtypeStruct((E,), jnp.float32),
            jax.ShapeDtypeStruct((NC, V, DP), jnp.int32),
            jax.ShapeDtypeStruct((NC, V, DP), jnp.int32),
        ],
        mesh=mesh,
        compiler_params=pltpu.CompilerParams(
            needs_layout_passes=False, use_tc_tiling_on_sc=False),
        scratch_types=[
            pltpu.VMEM((EW,), jnp.int32),
            pltpu.VMEM((EW,), jnp.int32),
            pltpu.VMEM((NBUF, CHB, DP), jnp.int32),
            pltpu.VMEM((NBUF, CHB, DP), jnp.int32),
            pltpu.VMEM((L * L,), jnp.float32),
            pltpu.VMEM((EW,), jnp.float32),
            pltpu.VMEM((2, PR, D), jnp.float32),
            pltpu.VMEM((PR, DP), jnp.int32),
            pltpu.SemaphoreType.DMA((NBUF,)),
            pltpu.SemaphoreType.DMA((2,)),
            pltpu.SemaphoreType.DMA,
        ],
    )(xu, xr, eli)


def kernel(x_user, x_restaurant, edge_label_index):
    out, _, _ = _run(x_user, x_restaurant,
                     edge_label_index.astype(jnp.int32))
    return out


# speedup vs baseline: 1.4508x; 1.3081x over previous
import jax
import jax.numpy as jnp
from jax import lax
from jax.experimental import pallas as pl
from jax.experimental.pallas import tpu as pltpu
from jax.experimental.pallas import tpu_sc as plsc

E = 320000
V = 10000
D = 128
DP = D // 2
NC = 2
NS = 16
L = 16
NW = NC * NS
EW = E // NW
CHB = 80
NCHUNK = EW // CHB
NBUF = 5
NGB = CHB // L
PR = 125
VS = V // NS
NPACK = VS // PR


def _body(xu, xr, eli, out, pu, pr_, idxu_v, idxr_v, urows, rrows, pbuf,
          out_all, pin, pout, sem, insem, outsem):
    c = lax.axis_index("c")
    s = lax.axis_index("s")
    wid = s * NC + c
    base_w = wid * EW

    icp0 = pltpu.async_copy(eli.at[0, pl.ds(base_w, EW)], idxu_v, sem.at[0])
    icp1 = pltpu.async_copy(eli.at[1, pl.ds(base_w, EW)], idxr_v, sem.at[1])

    def pack_table(src, dst):
        pltpu.async_copy(src.at[pl.ds(s * VS, PR)], pin.at[0], insem.at[0])

        def pack_step(pi, carry):
            pb = lax.rem(pi, 2)
            r0 = s * VS + pi * PR
            pltpu.make_async_copy(src.at[pl.ds(0, PR)], pin.at[pb],
                                  insem.at[pb]).wait()

            @pl.when(pi + 1 < NPACK)
            def _():
                pltpu.async_copy(src.at[pl.ds(r0 + PR, PR)],
                                 pin.at[1 - pb], insem.at[1 - pb])

            @pl.when(pi >= 1)
            def _():
                pltpu.make_async_copy(pout, dst.at[c, pl.ds(0, PR)],
                                      outsem).wait()

            def prow(i5, rcarry):
                for u in range(5):
                    i = i5 * 5 + u
                    for k in range(DP // L):
                        a = pin[pb, i, pl.ds(k * L, L)]
                        b = pin[pb, i, pl.ds(DP + k * L, L)]
                        w = plsc.pack(a, b, format=plsc.PackFormat.INTERLEAVED)
                        pout[i, pl.ds(k * L, L)] = plsc.bitcast(w, jnp.int32)
                return rcarry
            lax.fori_loop(0, PR // 5, prow, 0, unroll=False)
            pltpu.async_copy(pout, dst.at[c, pl.ds(r0, PR)], outsem)
            return carry
        lax.fori_loop(0, NPACK, pack_step, 0, unroll=False)
        pltpu.make_async_copy(pout, dst.at[c, pl.ds(0, PR)],
                              outsem).wait()

    pack_table(xu, pu)
    pack_table(xr, pr_)
    plsc.subcore_barrier()

    icp0.wait()
    icp1.wait()

    def issue(ci, b):
        pltpu.async_copy(
            pu.at[c].at[idxu_v.at[pl.ds(ci * CHB, CHB)]], urows.at[b],
            sem.at[b])
        pltpu.async_copy(
            pr_.at[c].at[idxr_v.at[pl.ds(ci * CHB, CHB)]], rrows.at[b],
            sem.at[b])

    for b in range(NBUF):
        issue(b, b)

    def chunk_body(ci, carry):
        b = lax.rem(ci, NBUF)
        pltpu.make_async_copy(pu.at[0, pl.ds(0, CHB)], urows.at[b],
                              sem.at[b]).wait()
        pltpu.make_async_copy(pu.at[0, pl.ds(0, CHB)], rrows.at[b],
                              sem.at[b]).wait()

        def group_body(g, gcarry):
            ps = []
            for i in range(L):
                e = g * L + i
                p = None
                for k in range(DP // L):
                    uv = plsc.bitcast(urows[b, e, pl.ds(k * L, L)], jnp.bfloat16)
                    rv = plsc.bitcast(rrows[b, e, pl.ds(k * L, L)], jnp.bfloat16)
                    t = uv * rv
                    p = t if p is None else p + t
                ps.append(p)
            for i in range(L):
                pa, pb = plsc.unpack(ps[i], format=plsc.PackFormat.INTERLEAVED)
                pbuf[pl.ds(i * L, L)] = pa + pb
            ebase = lax.iota(jnp.int32, L) * L
            acc = plsc.load_gather(pbuf, [ebase])
            for j in range(1, L):
                acc = acc + plsc.load_gather(pbuf, [ebase + j])
            out_all[pl.ds(ci * CHB + g * L, L)] = acc
            return gcarry

        lax.fori_loop(0, NGB, group_body, 0, unroll=False)

        @pl.when(ci < NCHUNK - NBUF)
        def _():
            issue(ci + NBUF, b)

        return carry

    lax.fori_loop(0, NCHUNK, chunk_body, 0, unroll=False)
    pltpu.sync_copy(out_all, out.at[pl.ds(base_w, EW)])


@jax.jit
def _run(xu, xr, eli):
    mesh = plsc.VectorSubcoreMesh(
        core_axis_name="c", subcore_axis_name="s", num_cores=NC,
        num_subcores=NS)
    return pl.kernel(
        _body,
        out_type=[
            jax.ShapeDtypeStruct((E,), jnp.float32),
            jax.ShapeDtypeStruct((NC, V, DP), jnp.int32),
            jax.ShapeDtypeStruct((NC, V, DP), jnp.int32),
        ],
        mesh=mesh,
        compiler_params=pltpu.CompilerParams(
            needs_layout_passes=False, use_tc_tiling_on_sc=False),
        scratch_types=[
            pltpu.VMEM((EW,), jnp.int32),
            pltpu.VMEM((EW,), jnp.int32),
            pltpu.VMEM((NBUF, CHB, DP), jnp.int32),
            pltpu.VMEM((NBUF, CHB, DP), jnp.int32),
            pltpu.VMEM((L * L,), jnp.float32),
            pltpu.VMEM((EW,), jnp.float32),
            pltpu.VMEM((2, PR, D), jnp.float32),
            pltpu.VMEM((PR, DP), jnp.int32),
            pltpu.SemaphoreType.DMA((NBUF,)),
            pltpu.SemaphoreType.DMA((2,)),
            pltpu.SemaphoreType.DMA,
        ],
    )(xu, xr, eli)


def kernel(x_user, x_restaurant, edge_label_index):
    out, _, _ = _run(x_user, x_restaurant,
                     edge_label_index.astype(jnp.int32))
    return out


# speedup vs baseline: 1.5622x; 1.0768x over previous
import jax
import jax.numpy as jnp
from jax import lax
from jax.experimental import pallas as pl
from jax.experimental.pallas import tpu as pltpu
from jax.experimental.pallas import tpu_sc as plsc

E = 320000
V = 10000
D = 128
DP = D // 2
NC = 2
NS = 16
L = 16
NW = NC * NS
EW = E // NW
CHB = 80
NCHUNK = EW // CHB
NBUF = 5
NGB = CHB // L
PR = 125
VS = V // NS
NPACK = VS // PR


def _body(xu, xr, eli, out, pu, pr_, idxu_v, idxr_v, urows, rrows, pbuf,
          out_all, pin, pout, sem, insem, outsem):
    c = lax.axis_index("c")
    s = lax.axis_index("s")
    wid = s * NC + c
    base_w = wid * EW

    icp0 = pltpu.async_copy(eli.at[0, pl.ds(base_w, EW)], idxu_v, sem.at[0])
    icp1 = pltpu.async_copy(eli.at[1, pl.ds(base_w, EW)], idxr_v, sem.at[1])

    def pack_table(src, dst):
        pltpu.async_copy(src.at[pl.ds(s * VS, PR)], pin.at[0], insem.at[0])

        def pack_step(pi, carry):
            pb = lax.rem(pi, 2)
            r0 = s * VS + pi * PR
            pltpu.make_async_copy(src.at[pl.ds(0, PR)], pin.at[pb],
                                  insem.at[pb]).wait()

            @pl.when(pi + 1 < NPACK)
            def _():
                pltpu.async_copy(src.at[pl.ds(r0 + PR, PR)],
                                 pin.at[1 - pb], insem.at[1 - pb])

            @pl.when(pi >= 1)
            def _():
                pltpu.make_async_copy(pout, dst.at[c, pl.ds(0, PR)],
                                      outsem).wait()

            def prow(i5, rcarry):
                ws = []
                for u in range(5):
                    i = i5 * 5 + u
                    for k in range(DP // L):
                        a = pin[pb, i, pl.ds(k * L, L)]
                        b = pin[pb, i, pl.ds(DP + k * L, L)]
                        ws.append(plsc.pack(
                            a, b, format=plsc.PackFormat.INTERLEAVED))
                for u in range(5):
                    i = i5 * 5 + u
                    for k in range(DP // L):
                        pout[i, pl.ds(k * L, L)] = plsc.bitcast(
                            ws[u * (DP // L) + k], jnp.int32)
                return rcarry
            lax.fori_loop(0, PR // 5, prow, 0, unroll=False)
            pltpu.async_copy(pout, dst.at[c, pl.ds(r0, PR)], outsem)
            return carry
        lax.fori_loop(0, NPACK, pack_step, 0, unroll=False)
        pltpu.make_async_copy(pout, dst.at[c, pl.ds(0, PR)],
                              outsem).wait()

    pack_table(xu, pu)
    pack_table(xr, pr_)
    plsc.subcore_barrier()

    icp0.wait()
    icp1.wait()

    def issue(ci, b):
        pltpu.async_copy(
            pu.at[c].at[idxu_v.at[pl.ds(ci * CHB, CHB)]], urows.at[b],
            sem.at[b])
        pltpu.async_copy(
            pr_.at[c].at[idxr_v.at[pl.ds(ci * CHB, CHB)]], rrows.at[b],
            sem.at[b])

    for b in range(NBUF):
        issue(b, b)

    def chunk_body(ci, carry):
        b = lax.rem(ci, NBUF)
        pltpu.make_async_copy(pu.at[0, pl.ds(0, CHB)], urows.at[b],
                              sem.at[b]).wait()
        pltpu.make_async_copy(pu.at[0, pl.ds(0, CHB)], rrows.at[b],
                              sem.at[b]).wait()

        def group_body(g, gcarry):
            ps = []
            for i in range(L):
                e = g * L + i
                p = None
                for k in range(DP // L):
                    uv = plsc.bitcast(urows[b, e, pl.ds(k * L, L)], jnp.bfloat16)
                    rv = plsc.bitcast(rrows[b, e, pl.ds(k * L, L)], jnp.bfloat16)
                    t = uv * rv
                    p = t if p is None else p + t
                ps.append(p)
            for i in range(L):
                pa, pb = plsc.unpack(ps[i], format=plsc.PackFormat.INTERLEAVED)
                pbuf[pl.ds(i * L, L)] = pa + pb
            ebase = lax.iota(jnp.int32, L) * L
            acc = plsc.load_gather(pbuf, [ebase])
            for j in range(1, L):
                acc = acc + plsc.load_gather(pbuf, [ebase + j])
            out_all[pl.ds(ci * CHB + g * L, L)] = acc
            return gcarry

        lax.fori_loop(0, NGB, group_body, 0, unroll=False)

        @pl.when(ci < NCHUNK - NBUF)
        def _():
            issue(ci + NBUF, b)

        return carry

    lax.fori_loop(0, NCHUNK, chunk_body, 0, unroll=False)
    pltpu.sync_copy(out_all, out.at[pl.ds(base_w, EW)])


@jax.jit
def _run(xu, xr, eli):
    mesh = plsc.VectorSubcoreMesh(
        core_axis_name="c", subcore_axis_name="s", num_cores=NC,
        num_subcores=NS)
    return pl.kernel(
        _body,
        out_type=[
            jax.ShapeDtypeStruct((E,), jnp.float32),
            jax.ShapeDtypeStruct((NC, V, DP), jnp.int32),
            jax.ShapeDtypeStruct((NC, V, DP), jnp.int32),
        ],
        mesh=mesh,
        compiler_params=pltpu.CompilerParams(
            needs_layout_passes=False, use_tc_tiling_on_sc=False),
        scratch_types=[
            pltpu.VMEM((EW,), jnp.int32),
            pltpu.VMEM((EW,), jnp.int32),
            pltpu.VMEM((NBUF, CHB, DP), jnp.int32),
            pltpu.VMEM((NBUF, CHB, DP), jnp.int32),
            pltpu.VMEM((L * L,), jnp.float32),
            pltpu.VMEM((EW,), jnp.float32),
            pltpu.VMEM((2, PR, D), jnp.float32),
            pltpu.VMEM((PR, DP), jnp.int32),
            pltpu.SemaphoreType.DMA((NBUF,)),
            pltpu.SemaphoreType.DMA((2,)),
            pltpu.SemaphoreType.DMA,
        ],
    )(xu, xr, eli)


def kernel(x_user, x_restaurant, edge_label_index):
    out, _, _ = _run(x_user, x_restaurant,
                     edge_label_index.astype(jnp.int32))
    return out
